# baseline (speedup 1.0000x reference)
import functools

import jax
import jax.numpy as jnp
from jax import lax
from jax.experimental import pallas as pl
from jax.experimental.pallas import tpu as pltpu
from jax.experimental.pallas import tpu_sc as plsc

_B, _N, _L, _H, _QH = 64, 576, 128, 2048, 1024
_K = 100
_KPAD = 128
_BQ = 8
_INV_SQRT2 = 0.7071067811865476


def _qfeat_body(lang_ref, valid_ref, wq_ref, bq_ref, qf_ref):
    rows = []
    for r in range(_BQ):
        v = valid_ref[r]
        d = jnp.maximum(jnp.sum(v), 1.0)
        qg = lax.dot_general(v, lang_ref[r], (((1,), (0,)), ((), ())),
                             preferred_element_type=jnp.float32) / d
        rows.append(qg)
    qg8 = jnp.concatenate(rows, axis=0)
    qf_ref[...] = lax.dot_general(qg8, wq_ref[...], (((1,), (1,)), ((), ())),
                                  preferred_element_type=jnp.float32) + bq_ref[...]


def _qfeat_call(lang, valid, wq, bq):
    return pl.pallas_call(
        _qfeat_body,
        grid=(_B // _BQ,),
        in_specs=[
            pl.BlockSpec((_BQ, _L, _H), lambda g: (g, 0, 0)),
            pl.BlockSpec((_BQ, 1, _L), lambda g: (g, 0, 0)),
            pl.BlockSpec((_QH, _H), lambda g: (0, 0)),
            pl.BlockSpec((1, _QH), lambda g: (0, 0)),
        ],
        out_specs=pl.BlockSpec((_BQ, _QH), lambda g: (g, 0)),
        out_shape=jax.ShapeDtypeStruct((_B, _QH), jnp.float32),
    )(lang, valid, wq, bq)


def _score_topk_body(img_ref, qf_ref, wv_ref, ws_ref, bv_ref, idx_ref):
    b = pl.program_id(0)
    img = img_ref[0]
    qf = qf_ref[0]

    vf = lax.dot_general(img, wv_ref[...], (((1,), (1,)), ((), ())),
                         preferred_element_type=jnp.float32) + bv_ref[...]
    h = jnp.tanh(vf + qf)
    g = h * 0.5 * (1.0 + lax.erf(h * _INV_SQRT2))
    s_col = lax.dot_general(g, ws_ref[...], (((1,), (1,)), ((), ())),
                            preferred_element_type=jnp.float32)

    ii = lax.broadcasted_iota(jnp.int32, (_N, _N), 0)
    jj = lax.broadcasted_iota(jnp.int32, (_N, _N), 1)
    ident = (ii == jj).astype(jnp.float32)
    s_row = lax.dot_general(s_col, ident, (((0,), (0,)), ((), ())),
                            preferred_element_type=jnp.float32)
    ones_1n = jnp.ones((1, _N), jnp.float32)
    s_mat = lax.dot_general(s_col, ones_1n, (((1,), (0,)), ((), ())),
                            preferred_element_type=jnp.float32)

    beats = (s_row > s_mat) | ((s_row == s_mat) & (jj < ii))
    rank = jnp.sum(beats.astype(jnp.float32), axis=1, keepdims=True)

    ones_1k = jnp.ones((1, _KPAD), jnp.float32)
    rank_mat = lax.dot_general(rank, ones_1k, (((1,), (0,)), ((), ())),
                               preferred_element_type=jnp.float32)
    flat_i = (lax.broadcasted_iota(jnp.int32, (_N, 1), 0)
              + b * _N).astype(jnp.float32)
    flat_mat = lax.dot_general(flat_i, ones_1k, (((1,), (0,)), ((), ())),
                               preferred_element_type=jnp.float32)
    p_row = lax.broadcasted_iota(jnp.int32, (1, _KPAD), 1).astype(jnp.float32)
    sel = (rank_mat == p_row).astype(jnp.float32)
    idx = jnp.sum(sel * flat_mat, axis=0, keepdims=True)
    idx_ref[0] = idx.astype(jnp.int32)


def _score_topk_call(img, qf, wv, ws, bv):
    return pl.pallas_call(
        _score_topk_body,
        grid=(_B,),
        in_specs=[
            pl.BlockSpec((1, _N, _H), lambda b: (b, 0, 0)),
            pl.BlockSpec((1, 1, _QH), lambda b: (b, 0, 0)),
            pl.BlockSpec((_QH, _H), lambda b: (0, 0)),
            pl.BlockSpec((1, _QH), lambda b: (0, 0)),
            pl.BlockSpec((1, _QH), lambda b: (0, 0)),
        ],
        out_specs=pl.BlockSpec((1, 1, _KPAD), lambda b: (b, 0, 0)),
        out_shape=jax.ShapeDtypeStruct((_B, 1, _KPAD), jnp.int32),
    )(img, qf, wv, ws, bv)


_NW = 32
_BPW = _B // _NW
_KP8 = 104
_IPW = _BPW * _KP8
_HQ = _H // 4


@functools.cache
def _gather_rows_kernel():
    @functools.partial(
        pl.kernel,
        out_type=jax.ShapeDtypeStruct((_B, _K, _H), jnp.float32),
        mesh=plsc.VectorSubcoreMesh(core_axis_name="c", subcore_axis_name="s"),
        scratch_types=[
            pltpu.VMEM((_IPW,), jnp.int32),
            pltpu.VMEM((_K, _HQ), jnp.float32),
            pltpu.VMEM((_K, _HQ), jnp.float32),
            pltpu.SemaphoreType.DMA,
            pltpu.SemaphoreType.DMA,
            pltpu.SemaphoreType.DMA,
            pltpu.SemaphoreType.DMA,
        ],
    )
    def _gather_rows(img_hbm, idx_hbm, out_hbm,
                     idx_v, buf0, buf1, g0, g1, w0, w1):
        table = img_hbm.reshape(_B * _N, _H)
        wid = lax.axis_index("s") * 2 + lax.axis_index("c")
        pltpu.sync_copy(idx_hbm.at[pl.ds(wid * _IPW, _IPW)], idx_v)
        bufs, gsems, wsems = (buf0, buf1), (g0, g1), (w0, w1)
        wh = [None, None]
        transfers = [(r, hq) for r in range(_BPW) for hq in range(_H // _HQ)]
        for i, (r, hq) in enumerate(transfers):
            bi = i & 1
            if wh[bi] is not None:
                wh[bi].wait()
            idxs = idx_v.at[pl.ds(r * _KP8, _K)]
            pltpu.async_copy(
                table.at[idxs, pl.ds(hq * _HQ, _HQ)], bufs[bi], gsems[bi]
            ).wait()
            wh[bi] = pltpu.async_copy(
                bufs[bi],
                out_hbm.at[wid * _BPW + r, :, pl.ds(hq * _HQ, _HQ)],
                wsems[bi])
        for h in wh:
            h.wait()

    return _gather_rows


def kernel(img_tokens, lang_tokens, lang_mask, Wq, bq, Wv, bv, Ws, bs):
    valid = (~lang_mask[:, 0, 0, :]).astype(jnp.float32).reshape(_B, 1, _L)
    qf = _qfeat_call(lang_tokens, valid, Wq, bq.reshape(1, _QH))
    flat_idx = _score_topk_call(
        img_tokens, qf.reshape(_B, 1, _QH), Wv, Ws,
        bv.reshape(1, _QH))
    idx = flat_idx[:, 0, :_KP8].reshape(-1)
    selected = _gather_rows_kernel()(img_tokens, idx)
    sel_mask = jnp.zeros((_B, 1, 1, _K), dtype=bool)
    return (selected, sel_mask)


# speedup vs baseline: 1.0508x; 1.0508x over previous
import functools

import jax
import jax.numpy as jnp
from jax import lax
from jax.experimental import pallas as pl
from jax.experimental.pallas import tpu as pltpu
from jax.experimental.pallas import tpu_sc as plsc

_B, _N, _L, _H, _QH = 64, 576, 128, 2048, 1024
_K = 100
_KPAD = 128
_BQ = 8
_INV_SQRT2 = 0.7071067811865476


def _qfeat_body(lang_ref, valid_ref, wq_ref, bq_ref, qf_ref):
    rows = []
    for r in range(_BQ):
        v = valid_ref[r]
        d = jnp.maximum(jnp.sum(v), 1.0)
        qg = lax.dot_general(v, lang_ref[r], (((1,), (0,)), ((), ())),
                             preferred_element_type=jnp.float32) / d
        rows.append(qg)
    qg8 = jnp.concatenate(rows, axis=0)
    qf_ref[...] = lax.dot_general(qg8, wq_ref[...], (((1,), (1,)), ((), ())),
                                  preferred_element_type=jnp.float32) + bq_ref[...]


def _qfeat_call(lang, valid, wq, bq):
    return pl.pallas_call(
        _qfeat_body,
        grid=(_B // _BQ,),
        in_specs=[
            pl.BlockSpec((_BQ, _L, _H), lambda g: (g, 0, 0)),
            pl.BlockSpec((_BQ, 1, _L), lambda g: (g, 0, 0)),
            pl.BlockSpec((_QH, _H), lambda g: (0, 0)),
            pl.BlockSpec((1, _QH), lambda g: (0, 0)),
        ],
        out_specs=pl.BlockSpec((_BQ, _QH), lambda g: (g, 0)),
        out_shape=jax.ShapeDtypeStruct((_B, _QH), jnp.float32),
    )(lang, valid, wq, bq)


def _score_topk_body(img_ref, qf_ref, wv_ref, ws_ref, bv_ref, idx_ref):
    b = pl.program_id(0)
    img = img_ref[0]
    qf = qf_ref[0]

    vf = lax.dot_general(img, wv_ref[...], (((1,), (1,)), ((), ())),
                         preferred_element_type=jnp.float32) + bv_ref[...]
    h = jnp.tanh(vf + qf)
    g = h * 0.5 * (1.0 + lax.erf(h * _INV_SQRT2))
    s_col = lax.dot_general(g, ws_ref[...], (((1,), (1,)), ((), ())),
                            preferred_element_type=jnp.float32)

    ii = lax.broadcasted_iota(jnp.int32, (_N, _N), 0)
    jj = lax.broadcasted_iota(jnp.int32, (_N, _N), 1)
    ident = (ii == jj).astype(jnp.float32)
    s_row = lax.dot_general(s_col, ident, (((0,), (0,)), ((), ())),
                            preferred_element_type=jnp.float32)
    ones_1n = jnp.ones((1, _N), jnp.float32)
    s_mat = lax.dot_general(s_col, ones_1n, (((1,), (0,)), ((), ())),
                            preferred_element_type=jnp.float32)

    beats = (s_row > s_mat) | ((s_row == s_mat) & (jj < ii))
    rank = jnp.sum(beats.astype(jnp.float32), axis=1, keepdims=True)

    ones_1k = jnp.ones((1, _KPAD), jnp.float32)
    rank_mat = lax.dot_general(rank, ones_1k, (((1,), (0,)), ((), ())),
                               preferred_element_type=jnp.float32)
    flat_i = (lax.broadcasted_iota(jnp.int32, (_N, 1), 0)
              + b * _N).astype(jnp.float32)
    flat_mat = lax.dot_general(flat_i, ones_1k, (((1,), (0,)), ((), ())),
                               preferred_element_type=jnp.float32)
    p_row = lax.broadcasted_iota(jnp.int32, (1, _KPAD), 1).astype(jnp.float32)
    sel = (rank_mat == p_row).astype(jnp.float32)
    idx = jnp.sum(sel * flat_mat, axis=0, keepdims=True)
    idx_ref[0] = idx.astype(jnp.int32)


def _score_topk_call(img, qf, wv, ws, bv):
    return pl.pallas_call(
        _score_topk_body,
        grid=(_B,),
        in_specs=[
            pl.BlockSpec((1, _N, _H), lambda b: (b, 0, 0)),
            pl.BlockSpec((1, 1, _QH), lambda b: (b, 0, 0)),
            pl.BlockSpec((_QH, _H), lambda b: (0, 0)),
            pl.BlockSpec((1, _QH), lambda b: (0, 0)),
            pl.BlockSpec((1, _QH), lambda b: (0, 0)),
        ],
        out_specs=pl.BlockSpec((1, 1, _KPAD), lambda b: (b, 0, 0)),
        out_shape=jax.ShapeDtypeStruct((_B, 1, _KPAD), jnp.int32),
    )(img, qf, wv, ws, bv)


_NW = 32
_BPW = _B // _NW
_KP8 = 104
_IPW = _BPW * _KP8
_CH = 24
_CHUNKS = ((0, 24), (24, 24), (48, 24), (72, 24), (96, 8))


@functools.cache
def _gather_rows_kernel():
    @functools.partial(
        pl.kernel,
        out_type=jax.ShapeDtypeStruct((_B, _KP8, _H), jnp.float32),
        mesh=plsc.VectorSubcoreMesh(core_axis_name="c", subcore_axis_name="s"),
        scratch_types=[
            pltpu.VMEM((_IPW,), jnp.int32),
            pltpu.VMEM((_CH, _H), jnp.float32),
            pltpu.VMEM((_CH, _H), jnp.float32),
            pltpu.SemaphoreType.DMA,
            pltpu.SemaphoreType.DMA,
            pltpu.SemaphoreType.DMA,
            pltpu.SemaphoreType.DMA,
        ],
    )
    def _gather_rows(img_hbm, idx_hbm, out_hbm,
                     idx_v, buf0, buf1, g0, g1, w0, w1):
        table = img_hbm.reshape(_B * _N, _H)
        wid = lax.axis_index("s") * 2 + lax.axis_index("c")
        pltpu.sync_copy(idx_hbm.at[pl.ds(wid * _IPW, _IPW)], idx_v)
        bufs, gsems, wsems = (buf0, buf1), (g0, g1), (w0, w1)
        wh = [None, None]
        transfers = [(r, k0, ln) for r in range(_BPW) for k0, ln in _CHUNKS]
        for i, (r, k0, ln) in enumerate(transfers):
            bi = i & 1
            if wh[bi] is not None:
                wh[bi].wait()
            dst = bufs[bi].at[pl.ds(0, ln)]
            pltpu.async_copy(
                table.at[idx_v.at[pl.ds(r * _KP8 + k0, ln)]], dst, gsems[bi]
            ).wait()
            wh[bi] = pltpu.async_copy(
                dst, out_hbm.at[wid * _BPW + r, pl.ds(k0, ln)], wsems[bi])
        for h in wh:
            h.wait()

    return _gather_rows


def kernel(img_tokens, lang_tokens, lang_mask, Wq, bq, Wv, bv, Ws, bs):
    valid = (~lang_mask[:, 0, 0, :]).astype(jnp.float32).reshape(_B, 1, _L)
    qf = _qfeat_call(lang_tokens, valid, Wq, bq.reshape(1, _QH))
    flat_idx = _score_topk_call(
        img_tokens, qf.reshape(_B, 1, _QH), Wv, Ws,
        bv.reshape(1, _QH))
    idx = flat_idx[:, 0, :_KP8].reshape(-1)
    sel_pad = _gather_rows_kernel()(img_tokens, idx)
    selected = sel_pad[:, :_K, :]
    sel_mask = jnp.zeros((_B, 1, 1, _K), dtype=bool)
    return (selected, sel_mask)
